# baseline (speedup 1.0000x reference)
import jax
import jax.numpy as jnp
from jax import lax
from jax.experimental import pallas as pl
from jax.experimental.pallas import tpu as pltpu
from jax.experimental.pallas import tpu_sc as plsc

_NV = 50000
_NR = 500
_NC = 2
_NS = 16
_NW = _NC * _NS
_L = 16
_ZC = 128
_K = 6


def _cdiv(a, b):
    return (a + b - 1) // b


def _make_edge_pass(e_pad, ch, head_split, t_src, t_dst, own, c_chunk):
    C = c_chunk
    rpt = _cdiv(_cdiv(own + 1, _NS), 8) * 8
    acc = rpt * _NS
    acc16 = acc // _L
    ept = e_pad // _NS
    n_chunks = ept // C
    nsc = n_chunks // _K
    ng = C // _L

    def body(src_h, dst_h, hl_h, hr_h, attb_h, zrows_h, zden_h,
             num_h, den_h,
             sidxb, didxb,
             sgatA, sgatB, dgatA, dgatB,
             accA, accB, accC, drowA, drowB, drowC,
             xjA, xjB, xiA, xiB, wstA, wstB, wstC,
             dnstA, dnstB, dnstC,
             attv_ref, num_sh, den_sh,
             gsemA, gsemB, ssemA, ssemB, ssemC):
        sgat, dgat = [sgatA, sgatB], [dgatA, dgatB]
        accv, drow = [accA, accB, accC], [drowA, drowB, drowC]
        xj, xi = [xjA, xjB], [xiA, xiB]
        wst, denst = [wstA, wstB, wstC], [dnstA, dnstB, dnstC]
        gsem, ssem = [gsemA, gsemB], [ssemA, ssemB, ssemC]
        cid = lax.axis_index("c")
        sid = lax.axis_index("s")
        iota = lax.iota(jnp.int32, _L)
        zf = jnp.zeros((_L,), jnp.float32)

        pltpu.sync_copy(attb_h.at[pl.ds(cid * ch, ch)], attv_ref)

        def dist_copy(n_rows, src_at, dst_at):
            nfull, rem = n_rows // _ZC, n_rows % _ZC

            def b(j, carry):
                @pl.when(j % _NS == sid)
                def _():
                    pltpu.sync_copy(src_at(j * _ZC, _ZC),
                                    dst_at(j * _ZC, _ZC))
                return carry

            lax.fori_loop(0, nfull, b, 0)
            if rem:
                @pl.when(nfull % _NS == sid)
                def _():
                    pltpu.sync_copy(src_at(nfull * _ZC, rem),
                                    dst_at(nfull * _ZC, rem))

        dist_copy(acc, lambda o, n: zrows_h.at[pl.ds(0, n)],
                  lambda o, n: num_sh.at[pl.ds(o, n)])
        dist_copy(acc16, lambda o, n: zden_h.at[pl.ds(0, n)],
                  lambda o, n: den_sh.at[pl.ds(o, n)])
        plsc.subcore_barrier()

        rows = [iota + g8 * _L for g8 in range(ng)]

        def transform(j, b2, b3):
            for g8 in range(ng):
                s = sidxb[pl.ds(j * C + g8 * _L, _L)]
                d = didxb[pl.ds(j * C + g8 * _L, _L)]
                if head_split:
                    sg = s + cid * t_src
                    dg = d + cid * t_dst
                    ai = d
                else:
                    lo = cid * own
                    inb = (d >= lo) & (d < lo + own)
                    ai = jnp.where(inb, d - lo, own)
                    sg = s
                    dg = d
                sgat[b2][pl.ds(g8 * _L, _L)] = sg
                dgat[b2][pl.ds(g8 * _L, _L)] = dg
                accv[b3][pl.ds(g8 * _L, _L)] = ai
                drow[b3][pl.ds(g8 * _L, _L)] = lax.shift_right_logical(ai, 4)

        def issue_gathers(b):
            pltpu.async_copy(hl_h.at[sgat[b]], xj[b], gsem[b])
            pltpu.async_copy(hr_h.at[dgat[b]], xi[b], gsem[b])

        def drain_gathers(b):
            pltpu.make_async_copy(zrows_h.at[pl.ds(0, C)], xj[b], gsem[b]).wait()
            pltpu.make_async_copy(zrows_h.at[pl.ds(0, C)], xi[b], gsem[b]).wait()

        def issue_scatters(b):
            pltpu.async_copy(wst[b], num_sh.at[accv[b]], ssem[b], add=True)
            pltpu.async_copy(denst[b], den_sh.at[drow[b]], ssem[b], add=True)


        def drain_scatters(b):
            pltpu.make_async_copy(zrows_h.at[pl.ds(0, C)], wst[b], ssem[b]).wait()
            pltpu.make_async_copy(zden_h.at[pl.ds(0, C)], denst[b], ssem[b]).wait()

        def compute(b2, b3):
            def abody(c, alphas):
                colc = jnp.full((_L,), c, jnp.int32)
                attv = plsc.load_gather(attv_ref, [colc, iota])
                out = []
                for g8 in range(ng):
                    a = plsc.load_gather(xj[b2], [rows[g8], colc])
                    bb = plsc.load_gather(xi[b2], [rows[g8], colc])
                    t = a + bb
                    t = jnp.maximum(t, t * 0.2)
                    out.append(alphas[g8] + attv * t)
                return tuple(out)

            alphas = lax.fori_loop(0, ch, abody,
                                   tuple(zf for _ in range(ng)), unroll=4)
            exs = [jnp.exp(a) for a in alphas]

            def zd(c, carry):
                colc = jnp.full((_L,), c, jnp.int32)
                for g8 in range(ng):
                    plsc.store_scatter(denst[b3], [rows[g8], colc], zf)
                return carry

            lax.fori_loop(0, _L, zd, 0, unroll=8)
            for g8 in range(ng):
                ai = accv[b3][pl.ds(g8 * _L, _L)]
                plsc.store_scatter(denst[b3], [rows[g8], ai & 15], exs[g8])

            def wbody(c, carry):
                colc = jnp.full((_L,), c, jnp.int32)
                for g8 in range(ng):
                    a = plsc.load_gather(xj[b2], [rows[g8], colc])
                    plsc.store_scatter(wst[b3], [rows[g8], colc], a * exs[g8])
                return carry

            lax.fori_loop(0, ch, wbody, 0, unroll=4)

        pltpu.sync_copy(src_h.at[pl.ds(sid * ept, _K * C)], sidxb)
        pltpu.sync_copy(dst_h.at[pl.ds(sid * ept, _K * C)], didxb)
        transform(0, 0, 0)
        issue_gathers(0)

        def sbody(s_i, carry):
            for k in range(_K):
                cur2, nxt2 = k % 2, (k + 1) % 2
                cur3, nxt3 = k % 3, (k + 1) % 3
                if k < 2:
                    @pl.when(s_i > 0)
                    def _():
                        drain_scatters(nxt3)
                else:
                    drain_scatters(nxt3)
                if k == _K - 1:
                    @pl.when(s_i + 1 < nsc)
                    def _():
                        e0n = sid * ept + (s_i + 1) * (_K * C)
                        pltpu.sync_copy(src_h.at[pl.ds(e0n, _K * C)], sidxb)
                        pltpu.sync_copy(dst_h.at[pl.ds(e0n, _K * C)], didxb)
                        transform(0, nxt2, nxt3)
                        issue_gathers(nxt2)
                else:
                    transform(k + 1, nxt2, nxt3)
                    issue_gathers(nxt2)
                drain_gathers(cur2)
                compute(cur2, cur3)
                issue_scatters(cur3)
            return carry

        lax.fori_loop(0, nsc, sbody, 0)
        drain_scatters(1)
        drain_scatters(2)
        plsc.subcore_barrier()

        dist_copy(acc, lambda o, n: num_sh.at[pl.ds(o, n)],
                  lambda o, n: num_h.at[pl.ds(cid * acc + o, n)])
        dist_copy(acc16, lambda o, n: den_sh.at[pl.ds(o, n)],
                  lambda o, n: den_h.at[pl.ds(cid * acc16 + o, n)])

    mesh = plsc.VectorSubcoreMesh(core_axis_name="c", subcore_axis_name="s",
                                  num_cores=_NC, num_subcores=_NS)
    i32, f32 = jnp.int32, jnp.float32
    fn = pl.kernel(
        body,
        out_type=(jax.ShapeDtypeStruct((2 * acc, ch), f32),
                  jax.ShapeDtypeStruct((2 * acc16, _L), f32)),
        mesh=mesh,
        compiler_params=pltpu.CompilerParams(use_tc_tiling_on_sc=False,
                                             needs_layout_passes=False),
        scratch_types=(
            [pltpu.VMEM((_K * C,), i32)] * 2
            + [pltpu.VMEM((C,), i32)] * 10
            + [pltpu.VMEM((C, ch), f32)] * 7
            + [pltpu.VMEM((C, _L), f32)] * 3
            + [pltpu.VMEM((ch, _L), f32),
               pltpu.VMEM_SHARED((acc, ch), f32),
               pltpu.VMEM_SHARED((acc16, _L), f32)]
            + [pltpu.SemaphoreType.DMA] * 5
        ),
    )
    return fn, acc, acc16


def _pad_edges(src, dst, n_dst, c_chunk):
    e = src.shape[0]
    gran = _NS * c_chunk * _K
    e_pad = _cdiv(e, gran) * gran
    pad = e_pad - e
    srcp = jnp.concatenate([src, jnp.zeros((pad,), jnp.int32)])
    dstp = jnp.concatenate([dst, jnp.full((pad,), n_dst, jnp.int32)])
    return srcp, dstp, e_pad


def _attb(att):
    h, ch = att.shape
    a2 = jnp.broadcast_to(att[:, :, None], (h, ch, _L))
    if h == 1:
        a2 = jnp.broadcast_to(a2, (2, ch, _L))
    return a2.reshape(2 * ch, _L)


def _lrelu(x):
    return jnp.maximum(x, x * 0.2)


def _elu(x):
    return jnp.where(x > 0, x, jnp.exp(jnp.minimum(x, 0.0)) - 1.0)


_BLK = 1000


def _k1v_body(x_ref, wlvv_ref, blvv_ref, wrvv_ref, brvv_ref, attvv_ref,
              wlvi_ref, blvi_ref, wriv_ref, briv_ref,
              hlvv_ref, hrvv_ref, exvv_ref, hlvi_ref, hriv_ref):
    x = x_ref[...]
    hlvv = jnp.dot(x, wlvv_ref[...].T, preferred_element_type=jnp.float32) + blvv_ref[...]
    hrvv = jnp.dot(x, wrvv_ref[...].T, preferred_element_type=jnp.float32) + brvv_ref[...]
    hlvi = jnp.dot(x, wlvi_ref[...].T, preferred_element_type=jnp.float32) + blvi_ref[...]
    hriv = jnp.dot(x, wriv_ref[...].T, preferred_element_type=jnp.float32) + briv_ref[...]
    att = attvv_ref[...]
    e = _lrelu(hlvv + hrvv)
    for h in range(2):
        hlvv_ref[h] = hlvv[:, 32 * h:32 * h + 32]
        hrvv_ref[h] = hrvv[:, 32 * h:32 * h + 32]
        hlvi_ref[h] = hlvi[:, 32 * h:32 * h + 32]
        hriv_ref[h] = hriv[:, 32 * h:32 * h + 32]
        exvv_ref[h] = jnp.exp(jnp.sum(e[:, 32 * h:32 * h + 32] * att[h], axis=-1))[:, None]


def _k1v(x_vehicle, Wl1_v2v, bl1_v2v, Wr1_v2v, br1_v2v, att1_v2v,
         Wl1_v2i, bl1_v2i, Wr1_i2v, br1_i2v):
    n = _NV
    grid = n // _BLK
    f32 = jnp.float32
    full = lambda s: pl.BlockSpec(s, lambda i: tuple(0 for _ in s))
    t3 = pl.BlockSpec((2, _BLK, 32), lambda i: (0, i, 0))
    t2 = pl.BlockSpec((2, _BLK, 1), lambda i: (0, i, 0))
    return pl.pallas_call(
        _k1v_body,
        grid=(grid,),
        in_specs=[pl.BlockSpec((_BLK, 6), lambda i: (i, 0)),
                  full((64, 6)), full((64,)), full((64, 6)), full((64,)),
                  full((2, 32)),
                  full((64, 6)), full((64,)), full((64, 6)), full((64,))],
        out_specs=[t3, t3, t2, t3, t3],
        out_shape=[jax.ShapeDtypeStruct((2, n, 32), f32),
                   jax.ShapeDtypeStruct((2, n, 32), f32),
                   jax.ShapeDtypeStruct((2, n, 1), f32),
                   jax.ShapeDtypeStruct((2, n, 32), f32),
                   jax.ShapeDtypeStruct((2, n, 32), f32)],
    )(x_vehicle, Wl1_v2v, bl1_v2v, Wr1_v2v, br1_v2v, att1_v2v,
      Wl1_v2i, bl1_v2i, Wr1_i2v, br1_i2v)


def _k1r_body(x_ref, wrvi_ref, brvi_ref, wliv_ref, bliv_ref,
              hrvi_ref, hliv_ref):
    x = x_ref[...]
    hrvi = jnp.dot(x, wrvi_ref[...].T, preferred_element_type=jnp.float32) + brvi_ref[...]
    hliv = jnp.dot(x, wliv_ref[...].T, preferred_element_type=jnp.float32) + bliv_ref[...]
    for h in range(2):
        hrvi_ref[h] = hrvi[:, 32 * h:32 * h + 32]
        hliv_ref[h] = hliv[:, 32 * h:32 * h + 32]


def _k1r(x_rsu, Wr1_v2i, br1_v2i, Wl1_i2v, bl1_i2v):
    f32 = jnp.float32
    return pl.pallas_call(
        _k1r_body,
        out_shape=[jax.ShapeDtypeStruct((2, _NR, 32), f32),
                   jax.ShapeDtypeStruct((2, _NR, 32), f32)],
    )(x_rsu, Wr1_v2i, br1_v2i, Wl1_i2v, bl1_i2v)


def _k2v_body(numvv_ref, denvv_ref, exvv_ref, hlvv_ref,
              numiv_ref, deniv_ref,
              b1vv_ref, b1iv_ref,
              wl2vv_ref, bl2vv_ref, wr2vv_ref, br2vv_ref, att2vv_ref,
              wl2vi_ref, bl2vi_ref, wr2iv_ref, br2iv_ref,
              hl2vv_ref, hr2vv_ref, ex2vv_ref, hl2vi_ref, hr2iv_ref):
    parts = []
    for h in range(2):
        ex = exvv_ref[h]
        den = denvv_ref[h] + ex
        v = (numvv_ref[h] + ex * hlvv_ref[h]) / (den + 1e-16)
        parts.append(v)
    v1 = jnp.concatenate(parts, axis=1) + b1vv_ref[...]
    parts = []
    for h in range(2):
        parts.append(numiv_ref[h] / (deniv_ref[h] + 1e-16))
    v1b = jnp.concatenate(parts, axis=1) + b1iv_ref[...]
    veh = _elu(v1 + v1b)
    hl2vv = jnp.dot(veh, wl2vv_ref[...].T, preferred_element_type=jnp.float32) + bl2vv_ref[...]
    hr2vv = jnp.dot(veh, wr2vv_ref[...].T, preferred_element_type=jnp.float32) + br2vv_ref[...]
    hl2vv_ref[...] = hl2vv
    hr2vv_ref[...] = hr2vv
    hl2vi_ref[...] = jnp.dot(veh, wl2vi_ref[...].T, preferred_element_type=jnp.float32) + bl2vi_ref[...]
    hr2iv_ref[...] = jnp.dot(veh, wr2iv_ref[...].T, preferred_element_type=jnp.float32) + br2iv_ref[...]
    ex2vv_ref[...] = jnp.exp(jnp.sum(_lrelu(hl2vv + hr2vv) * att2vv_ref[0], axis=-1))[:, None]


def _k2v(numvv, denvv, exvv, hlvv, numiv, deniv, b1vv, b1iv,
         Wl2_v2v, bl2_v2v, Wr2_v2v, br2_v2v, att2_v2v,
         Wl2_v2i, bl2_v2i, Wr2_i2v, br2_i2v):
    n = _NV
    grid = n // _BLK
    f32 = jnp.float32
    full = lambda s: pl.BlockSpec(s, lambda i: tuple(0 for _ in s))
    t3 = pl.BlockSpec((2, _BLK, 32), lambda i: (0, i, 0))
    t2 = pl.BlockSpec((2, _BLK, 1), lambda i: (0, i, 0))
    m = pl.BlockSpec((_BLK, 64), lambda i: (i, 0))
    m1 = pl.BlockSpec((_BLK, 1), lambda i: (i, 0))
    return pl.pallas_call(
        _k2v_body,
        grid=(grid,),
        in_specs=[t3, t2, t2, t3, t3, t2,
                  full((64,)), full((64,)),
                  full((64, 64)), full((64,)), full((64, 64)), full((64,)),
                  full((1, 64)),
                  full((64, 64)), full((64,)), full((64, 64)), full((64,))],
        out_specs=[m, m, m1, m, m],
        out_shape=[jax.ShapeDtypeStruct((n, 64), f32),
                   jax.ShapeDtypeStruct((n, 64), f32),
                   jax.ShapeDtypeStruct((n, 1), f32),
                   jax.ShapeDtypeStruct((n, 64), f32),
                   jax.ShapeDtypeStruct((n, 64), f32)],
    )(numvv, denvv, exvv, hlvv, numiv, deniv, b1vv, b1iv,
      Wl2_v2v, bl2_v2v, Wr2_v2v, br2_v2v, att2_v2v,
      Wl2_v2i, bl2_v2i, Wr2_i2v, br2_i2v)


def _k2r_body(numvi_ref, denvi_ref, b1vi_ref,
              wr2vi_ref, br2vi_ref, wl2iv_ref, bl2iv_ref,
              hr2vi_ref, hl2iv_ref):
    parts = []
    for h in range(2):
        parts.append(numvi_ref[h] / (denvi_ref[h] + 1e-16))
    r1 = jnp.concatenate(parts, axis=1) + b1vi_ref[...]
    rsu = _elu(r1)
    hr2vi_ref[...] = jnp.dot(rsu, wr2vi_ref[...].T, preferred_element_type=jnp.float32) + br2vi_ref[...]
    hl2iv_ref[...] = jnp.dot(rsu, wl2iv_ref[...].T, preferred_element_type=jnp.float32) + bl2iv_ref[...]


def _k2r(numvi, denvi, b1vi, Wr2_v2i, br2_v2i, Wl2_i2v, bl2_i2v):
    f32 = jnp.float32
    return pl.pallas_call(
        _k2r_body,
        out_shape=[jax.ShapeDtypeStruct((_NR, 64), f32),
                   jax.ShapeDtypeStruct((_NR, 64), f32)],
    )(numvi, denvi, b1vi, Wr2_v2i, br2_v2i, Wl2_i2v, bl2_i2v)


def _ln(x, g, b):
    mu = jnp.mean(x, axis=-1, keepdims=True)
    var = jnp.mean((x - mu) ** 2, axis=-1, keepdims=True)
    return (x - mu) * lax.rsqrt(var + 1e-5) * g + b


def _k3r_body(numvi_ref, denvi_ref, hl2vi5_ref, hr2vi_ref, attvi_ref,
              b2vi_ref, hl2iv_ref, hr2iv5_ref, attiv_ref, g_ref, b_ref,
              rsu_ref, exiv_ref):
    exvi = jnp.exp(jnp.sum(_lrelu(hl2vi5_ref[...] + hr2vi_ref[...]) * attvi_ref[0], axis=-1))[:, None]
    den = denvi_ref[...] + exvi
    r2 = (numvi_ref[...] + exvi * hl2vi5_ref[...]) / (den + 1e-16) + b2vi_ref[...]
    rsu_ref[...] = _ln(r2, g_ref[...], b_ref[...])
    exiv_ref[...] = jnp.exp(jnp.sum(_lrelu(hl2iv_ref[...] + hr2iv5_ref[...]) * attiv_ref[0], axis=-1))[:, None]


def _k3r(numvi, denvi, hl2vi5, hr2vi, att2_v2i, b2_v2i,
         hl2iv, hr2iv5, att2_i2v, g_rsu, be_rsu):
    f32 = jnp.float32
    return pl.pallas_call(
        _k3r_body,
        out_shape=[jax.ShapeDtypeStruct((_NR, 64), f32),
                   jax.ShapeDtypeStruct((_NR, 1), f32)],
    )(numvi, denvi, hl2vi5, hr2vi, att2_v2i, b2_v2i,
      hl2iv, hr2iv5, att2_i2v, g_rsu, be_rsu)


def _k3v_body(numvv_ref, denvv_ref, exvv_ref, hlvv_ref,
              numiv_ref, deniv_ref, exiv_ref, hliv_ref,
              b2vv_ref, b2iv_ref, g_ref, b_ref, out_ref):
    ex = exvv_ref[...]
    den = denvv_ref[...] + ex
    v2 = (numvv_ref[...] + ex * hlvv_ref[...]) / (den + 1e-16) + b2vv_ref[...]
    exb = exiv_ref[...]
    denb = deniv_ref[...] + exb
    v2b = (numiv_ref[...] + exb * hliv_ref[...]) / (denb + 1e-16) + b2iv_ref[...]
    out_ref[...] = _ln(v2 + v2b, g_ref[...], b_ref[...])


def _k3v(numvv, denvv, exvv, hlvv, numiv, deniv, exiv, hliv,
         b2vv, b2iv, g_veh, be_veh):
    n = _NV
    grid = n // _BLK
    f32 = jnp.float32
    full = lambda s: pl.BlockSpec(s, lambda i: tuple(0 for _ in s))
    m = pl.BlockSpec((_BLK, 64), lambda i: (i, 0))
    m1 = pl.BlockSpec((_BLK, 1), lambda i: (i, 0))
    return pl.pallas_call(
        _k3v_body,
        grid=(grid,),
        in_specs=[m, m1, m1, m, m, m1, m1, m,
                  full((64,)), full((64,)), full((64,)), full((64,))],
        out_specs=m,
        out_shape=jax.ShapeDtypeStruct((n, 64), f32),
    )(numvv, denvv, exvv, hlvv, numiv, deniv, exiv, hliv,
      b2vv, b2iv, g_veh, be_veh)


def _pad_row(t):
    h, n, ch = t.shape
    return jnp.pad(t, ((0, 0), (0, 1), (0, 0))).reshape(h * (n + 1), ch)


def _den_slice(den, acc16, own):
    return den.reshape(2, acc16 * _L)[:, :own, None]


def kernel(x_vehicle, x_rsu, edge_index_v2v, v2i_src, v2i_dst, i2v_src, i2v_dst,
           Wl1_v2v, bl1_v2v, Wr1_v2v, br1_v2v, att1_v2v, b1_v2v,
           Wl1_v2i, bl1_v2i, Wr1_v2i, br1_v2i, att1_v2i, b1_v2i,
           Wl1_i2v, bl1_i2v, Wr1_i2v, br1_i2v, att1_i2v, b1_i2v,
           Wl2_v2v, bl2_v2v, Wr2_v2v, br2_v2v, att2_v2v, b2_v2v,
           Wl2_v2i, bl2_v2i, Wr2_v2i, br2_v2i, att2_v2i, b2_v2i,
           Wl2_i2v, bl2_i2v, Wr2_i2v, br2_i2v, att2_i2v, b2_i2v,
           g_veh, be_veh, g_rsu, be_rsu):
    f32 = jnp.float32
    s, d = edge_index_v2v[0], edge_index_v2v[1]
    c32, c64 = 48, 32
    sp_vv1, dp_vv1, ep_vv1 = _pad_edges(s, d, _NV, c32)
    sp_vi1, dp_vi1, ep_vi1 = _pad_edges(v2i_src, v2i_dst, _NR, c32)
    sp_iv1, dp_iv1, ep_iv1 = _pad_edges(i2v_src, i2v_dst, _NV, c32)
    sp_vv2, dp_vv2, ep_vv2 = _pad_edges(s, d, _NV, c64)
    sp_vi2, dp_vi2, ep_vi2 = _pad_edges(v2i_src, v2i_dst, _NR, c64)
    sp_iv2, dp_iv2, ep_iv2 = _pad_edges(i2v_src, i2v_dst, _NV, c64)

    z32 = jnp.zeros((_ZC, 32), f32)
    z64 = jnp.zeros((_ZC, 64), f32)
    zd16 = jnp.zeros((_ZC, _L), f32)

    hlvv, hrvv, exvv, hlvi, hriv = _k1v(
        x_vehicle, Wl1_v2v, bl1_v2v, Wr1_v2v, br1_v2v, att1_v2v,
        Wl1_v2i, bl1_v2i, Wr1_i2v, br1_i2v)
    hrvi, hliv = _k1r(x_rsu, Wr1_v2i, br1_v2i, Wl1_i2v, bl1_i2v)

    f_vv1, acc_vv1, a16_vv1 = _make_edge_pass(ep_vv1, 32, True, _NV, _NV + 1, _NV, c32)
    f_vi1, acc_vi1, a16_vi1 = _make_edge_pass(ep_vi1, 32, True, _NV, _NR + 1, _NR, c32)
    f_iv1, acc_iv1, a16_iv1 = _make_edge_pass(ep_iv1, 32, True, _NR, _NV + 1, _NV, c32)

    num1vv, den1vv = f_vv1(sp_vv1, dp_vv1, hlvv.reshape(2 * _NV, 32),
                           _pad_row(hrvv), _attb(att1_v2v), z32, zd16)
    num1vi, den1vi = f_vi1(sp_vi1, dp_vi1, hlvi.reshape(2 * _NV, 32),
                           _pad_row(hrvi), _attb(att1_v2i), z32, zd16)
    num1iv, den1iv = f_iv1(sp_iv1, dp_iv1, hliv.reshape(2 * _NR, 32),
                           _pad_row(hriv), _attb(att1_i2v), z32, zd16)

    num1vv_s = num1vv.reshape(2, acc_vv1, 32)[:, :_NV]
    den1vv_s = _den_slice(den1vv, a16_vv1, _NV)
    num1iv_s = num1iv.reshape(2, acc_iv1, 32)[:, :_NV]
    den1iv_s = _den_slice(den1iv, a16_iv1, _NV)
    num1vi_s = num1vi.reshape(2, acc_vi1, 32)[:, :_NR]
    den1vi_s = _den_slice(den1vi, a16_vi1, _NR)

    hl2vv, hr2vv, ex2vv, hl2vi, hr2iv = _k2v(
        num1vv_s, den1vv_s, exvv, hlvv, num1iv_s, den1iv_s, b1_v2v, b1_i2v,
        Wl2_v2v, bl2_v2v, Wr2_v2v, br2_v2v, att2_v2v,
        Wl2_v2i, bl2_v2i, Wr2_i2v, br2_i2v)
    hr2vi, hl2iv = _k2r(num1vi_s, den1vi_s, b1_v2i,
                        Wr2_v2i, br2_v2i, Wl2_i2v, bl2_i2v)

    f_vv2, acc_vv2, a16_vv2 = _make_edge_pass(ep_vv2, 64, False, _NV, _NV + 1, _NV // 2, c64)
    f_vi2, acc_vi2, a16_vi2 = _make_edge_pass(ep_vi2, 64, False, _NV, _NR + 1, _NR // 2, c64)
    f_iv2, acc_iv2, a16_iv2 = _make_edge_pass(ep_iv2, 64, False, _NR, _NV + 1, _NV // 2, c64)

    hr2vv_p = jnp.pad(hr2vv, ((0, 1), (0, 0)))
    hr2vi_p = jnp.pad(hr2vi, ((0, 1), (0, 0)))
    hr2iv_p = jnp.pad(hr2iv, ((0, 1), (0, 0)))
    num2vv, den2vv = f_vv2(sp_vv2, dp_vv2, hl2vv, hr2vv_p, _attb(att2_v2v), z64, zd16)
    num2vi, den2vi = f_vi2(sp_vi2, dp_vi2, hl2vi, hr2vi_p, _attb(att2_v2i), z64, zd16)
    num2iv, den2iv = f_iv2(sp_iv2, dp_iv2, hl2iv, hr2iv_p, _attb(att2_i2v), z64, zd16)

    h2 = _NV // 2
    num2vv_s = num2vv.reshape(2, acc_vv2, 64)[:, :h2].reshape(_NV, 64)
    den2vv_s = _den_slice(den2vv, a16_vv2, h2).reshape(_NV, 1)
    num2iv_s = num2iv.reshape(2, acc_iv2, 64)[:, :h2].reshape(_NV, 64)
    den2iv_s = _den_slice(den2iv, a16_iv2, h2).reshape(_NV, 1)
    hr2n = _NR // 2
    num2vi_s = num2vi.reshape(2, acc_vi2, 64)[:, :hr2n].reshape(_NR, 64)
    den2vi_s = _den_slice(den2vi, a16_vi2, hr2n).reshape(_NR, 1)

    rsu_out, exiv = _k3r(num2vi_s, den2vi_s, hl2vi[:_NR], hr2vi, att2_v2i,
                         b2_v2i, hl2iv, hr2iv[:_NR], att2_i2v, g_rsu, be_rsu)
    exiv_pad = jnp.pad(exiv, ((0, _NV - _NR), (0, 0)))
    hl2iv_pad = jnp.pad(hl2iv, ((0, _NV - _NR), (0, 0)))
    veh_out = _k3v(num2vv_s, den2vv_s, ex2vv, hl2vv,
                   num2iv_s, den2iv_s, exiv_pad, hl2iv_pad,
                   b2_v2v, b2_i2v, g_veh, be_veh)
    return veh_out, rsu_out


# speedup vs baseline: 3.3035x; 3.3035x over previous
import jax
import jax.numpy as jnp
from jax import lax
from jax.experimental import pallas as pl
from jax.experimental.pallas import tpu as pltpu
from jax.experimental.pallas import tpu_sc as plsc

_NV = 50000
_NR = 500
_NC = 2
_NS = 16
_NW = _NC * _NS
_L = 16
_ZC = 128
_K = 6


def _cdiv(a, b):
    return (a + b - 1) // b


def _make_edge_pass(e_pad, ch, head_split, t_src, t_dst, own, c_chunk):
    C = c_chunk
    rpt = _cdiv(_cdiv(own + 1, _NS), 8) * 8
    acc = rpt * _NS
    acc16 = acc // _L
    ept = e_pad // _NS
    n_chunks = ept // C
    nsc = n_chunks // _K
    ng = C // _L

    def body(src_h, dst_h, hl_h, hr_h, attb_h, zrows_h, zden_h,
             num_h, den_h,
             sidxb, didxb,
             sgatA, sgatB, dgatA, dgatB,
             accA, accB, accC, drowA, drowB, drowC,
             xjA, xjB, xiA, xiB, wstA, wstB, wstC,
             dnstA, dnstB, dnstC,
             attv_ref, num_sh, den_sh,
             gsemA, gsemB, ssemA, ssemB, ssemC):
        sgat, dgat = [sgatA, sgatB], [dgatA, dgatB]
        accv, drow = [accA, accB, accC], [drowA, drowB, drowC]
        xj, xi = [xjA, xjB], [xiA, xiB]
        wst, denst = [wstA, wstB, wstC], [dnstA, dnstB, dnstC]
        gsem, ssem = [gsemA, gsemB], [ssemA, ssemB, ssemC]
        cid = lax.axis_index("c")
        sid = lax.axis_index("s")
        iota = lax.iota(jnp.int32, _L)
        zf = jnp.zeros((_L,), jnp.float32)

        pltpu.sync_copy(attb_h.at[pl.ds(cid * ch, ch)], attv_ref)

        def dist_copy(n_rows, src_at, dst_at):
            nfull, rem = n_rows // _ZC, n_rows % _ZC

            def b(j, carry):
                @pl.when(j % _NS == sid)
                def _():
                    pltpu.sync_copy(src_at(j * _ZC, _ZC),
                                    dst_at(j * _ZC, _ZC))
                return carry

            lax.fori_loop(0, nfull, b, 0)
            if rem:
                @pl.when(nfull % _NS == sid)
                def _():
                    pltpu.sync_copy(src_at(nfull * _ZC, rem),
                                    dst_at(nfull * _ZC, rem))

        dist_copy(acc, lambda o, n: zrows_h.at[pl.ds(0, n)],
                  lambda o, n: num_sh.at[pl.ds(o, n)])
        dist_copy(acc16, lambda o, n: zden_h.at[pl.ds(0, n)],
                  lambda o, n: den_sh.at[pl.ds(o, n)])
        plsc.subcore_barrier()

        rows = [iota + g8 * _L for g8 in range(ng)]

        def transform(j, b2, b3):
            for g8 in range(ng):
                s = sidxb[pl.ds(j * C + g8 * _L, _L)]
                d = didxb[pl.ds(j * C + g8 * _L, _L)]
                if head_split:
                    sg = s + cid * t_src
                    dg = d + cid * t_dst
                    ai = d
                else:
                    lo = cid * own
                    inb = (d >= lo) & (d < lo + own)
                    ai = jnp.where(inb, d - lo, own)
                    sg = s
                    dg = d
                sgat[b2][pl.ds(g8 * _L, _L)] = sg
                dgat[b2][pl.ds(g8 * _L, _L)] = dg
                accv[b3][pl.ds(g8 * _L, _L)] = ai
                drow[b3][pl.ds(g8 * _L, _L)] = lax.shift_right_logical(ai, 4)

        def issue_gathers(b):
            pltpu.async_copy(hl_h.at[sgat[b]], xj[b], gsem[b])
            pltpu.async_copy(hr_h.at[dgat[b]], xi[b], gsem[b])

        def drain_gathers(b):
            pltpu.make_async_copy(zrows_h.at[pl.ds(0, C)], xj[b], gsem[b]).wait()
            pltpu.make_async_copy(zrows_h.at[pl.ds(0, C)], xi[b], gsem[b]).wait()

        def issue_scatters(b):
            pltpu.async_copy(wst[b], num_sh.at[accv[b]], ssem[b], add=True)
            pltpu.async_copy(denst[b], den_sh.at[drow[b]], ssem[b], add=True)


        def drain_scatters(b):
            pltpu.make_async_copy(zrows_h.at[pl.ds(0, C)], wst[b], ssem[b]).wait()
            pltpu.make_async_copy(zden_h.at[pl.ds(0, C)], denst[b], ssem[b]).wait()

        def compute(b2, b3):
            def abody(c, alphas):
                colc = (iota + c) & (ch - 1)
                attv = plsc.load_gather(attv_ref, [colc, iota])
                out = []
                for g8 in range(ng):
                    a = plsc.load_gather(xj[b2], [rows[g8], colc])
                    bb = plsc.load_gather(xi[b2], [rows[g8], colc])
                    t = a + bb
                    t = jnp.maximum(t, t * 0.2)
                    out.append(alphas[g8] + attv * t)
                return tuple(out)

            alphas = lax.fori_loop(0, ch, abody,
                                   tuple(zf for _ in range(ng)), unroll=4)
            exs = [jnp.exp(a) for a in alphas]

            def zd(c, carry):
                colc = (iota + c) & (_L - 1)
                for g8 in range(ng):
                    plsc.store_scatter(denst[b3], [rows[g8], colc], zf)
                return carry

            lax.fori_loop(0, _L, zd, 0, unroll=8)
            for g8 in range(ng):
                ai = accv[b3][pl.ds(g8 * _L, _L)]
                plsc.store_scatter(denst[b3], [rows[g8], ai & 15], exs[g8])

            def wbody(c, carry):
                colc = (iota + c) & (ch - 1)
                for g8 in range(ng):
                    a = plsc.load_gather(xj[b2], [rows[g8], colc])
                    plsc.store_scatter(wst[b3], [rows[g8], colc], a * exs[g8])
                return carry

            lax.fori_loop(0, ch, wbody, 0, unroll=4)

        pltpu.sync_copy(src_h.at[pl.ds(sid * ept, _K * C)], sidxb)
        pltpu.sync_copy(dst_h.at[pl.ds(sid * ept, _K * C)], didxb)
        transform(0, 0, 0)
        issue_gathers(0)

        def sbody(s_i, carry):
            for k in range(_K):
                cur2, nxt2 = k % 2, (k + 1) % 2
                cur3, nxt3 = k % 3, (k + 1) % 3
                if k < 2:
                    @pl.when(s_i > 0)
                    def _():
                        drain_scatters(nxt3)
                else:
                    drain_scatters(nxt3)
                if k == _K - 1:
                    @pl.when(s_i + 1 < nsc)
                    def _():
                        e0n = sid * ept + (s_i + 1) * (_K * C)
                        pltpu.sync_copy(src_h.at[pl.ds(e0n, _K * C)], sidxb)
                        pltpu.sync_copy(dst_h.at[pl.ds(e0n, _K * C)], didxb)
                        transform(0, nxt2, nxt3)
                        issue_gathers(nxt2)
                else:
                    transform(k + 1, nxt2, nxt3)
                    issue_gathers(nxt2)
                drain_gathers(cur2)
                compute(cur2, cur3)
                issue_scatters(cur3)
            return carry

        lax.fori_loop(0, nsc, sbody, 0)
        drain_scatters(1)
        drain_scatters(2)
        plsc.subcore_barrier()

        dist_copy(acc, lambda o, n: num_sh.at[pl.ds(o, n)],
                  lambda o, n: num_h.at[pl.ds(cid * acc + o, n)])
        dist_copy(acc16, lambda o, n: den_sh.at[pl.ds(o, n)],
                  lambda o, n: den_h.at[pl.ds(cid * acc16 + o, n)])

    mesh = plsc.VectorSubcoreMesh(core_axis_name="c", subcore_axis_name="s",
                                  num_cores=_NC, num_subcores=_NS)
    i32, f32 = jnp.int32, jnp.float32
    fn = pl.kernel(
        body,
        out_type=(jax.ShapeDtypeStruct((2 * acc, ch), f32),
                  jax.ShapeDtypeStruct((2 * acc16, _L), f32)),
        mesh=mesh,
        compiler_params=pltpu.CompilerParams(use_tc_tiling_on_sc=False,
                                             needs_layout_passes=False),
        scratch_types=(
            [pltpu.VMEM((_K * C,), i32)] * 2
            + [pltpu.VMEM((C,), i32)] * 10
            + [pltpu.VMEM((C, ch), f32)] * 7
            + [pltpu.VMEM((C, _L), f32)] * 3
            + [pltpu.VMEM((ch, _L), f32),
               pltpu.VMEM_SHARED((acc, ch), f32),
               pltpu.VMEM_SHARED((acc16, _L), f32)]
            + [pltpu.SemaphoreType.DMA] * 5
        ),
    )
    return fn, acc, acc16


def _pad_edges(src, dst, n_dst, c_chunk):
    e = src.shape[0]
    gran = _NS * c_chunk * _K
    e_pad = _cdiv(e, gran) * gran
    pad = e_pad - e
    srcp = jnp.concatenate([src, jnp.zeros((pad,), jnp.int32)])
    dstp = jnp.concatenate([dst, jnp.full((pad,), n_dst, jnp.int32)])
    return srcp, dstp, e_pad


def _attb(att):
    h, ch = att.shape
    a2 = jnp.broadcast_to(att[:, :, None], (h, ch, _L))
    if h == 1:
        a2 = jnp.broadcast_to(a2, (2, ch, _L))
    return a2.reshape(2 * ch, _L)


def _lrelu(x):
    return jnp.maximum(x, x * 0.2)


def _elu(x):
    return jnp.where(x > 0, x, jnp.exp(jnp.minimum(x, 0.0)) - 1.0)


_BLK = 1000


def _k1v_body(x_ref, wlvv_ref, blvv_ref, wrvv_ref, brvv_ref, attvv_ref,
              wlvi_ref, blvi_ref, wriv_ref, briv_ref,
              hlvv_ref, hrvv_ref, exvv_ref, hlvi_ref, hriv_ref):
    x = x_ref[...]
    hlvv = jnp.dot(x, wlvv_ref[...].T, preferred_element_type=jnp.float32) + blvv_ref[...]
    hrvv = jnp.dot(x, wrvv_ref[...].T, preferred_element_type=jnp.float32) + brvv_ref[...]
    hlvi = jnp.dot(x, wlvi_ref[...].T, preferred_element_type=jnp.float32) + blvi_ref[...]
    hriv = jnp.dot(x, wriv_ref[...].T, preferred_element_type=jnp.float32) + briv_ref[...]
    att = attvv_ref[...]
    e = _lrelu(hlvv + hrvv)
    for h in range(2):
        hlvv_ref[h] = hlvv[:, 32 * h:32 * h + 32]
        hrvv_ref[h] = hrvv[:, 32 * h:32 * h + 32]
        hlvi_ref[h] = hlvi[:, 32 * h:32 * h + 32]
        hriv_ref[h] = hriv[:, 32 * h:32 * h + 32]
        exvv_ref[h] = jnp.exp(jnp.sum(e[:, 32 * h:32 * h + 32] * att[h], axis=-1))[:, None]


def _k1v(x_vehicle, Wl1_v2v, bl1_v2v, Wr1_v2v, br1_v2v, att1_v2v,
         Wl1_v2i, bl1_v2i, Wr1_i2v, br1_i2v):
    n = _NV
    grid = n // _BLK
    f32 = jnp.float32
    full = lambda s: pl.BlockSpec(s, lambda i: tuple(0 for _ in s))
    t3 = pl.BlockSpec((2, _BLK, 32), lambda i: (0, i, 0))
    t2 = pl.BlockSpec((2, _BLK, 1), lambda i: (0, i, 0))
    return pl.pallas_call(
        _k1v_body,
        grid=(grid,),
        in_specs=[pl.BlockSpec((_BLK, 6), lambda i: (i, 0)),
                  full((64, 6)), full((64,)), full((64, 6)), full((64,)),
                  full((2, 32)),
                  full((64, 6)), full((64,)), full((64, 6)), full((64,))],
        out_specs=[t3, t3, t2, t3, t3],
        out_shape=[jax.ShapeDtypeStruct((2, n, 32), f32),
                   jax.ShapeDtypeStruct((2, n, 32), f32),
                   jax.ShapeDtypeStruct((2, n, 1), f32),
                   jax.ShapeDtypeStruct((2, n, 32), f32),
                   jax.ShapeDtypeStruct((2, n, 32), f32)],
    )(x_vehicle, Wl1_v2v, bl1_v2v, Wr1_v2v, br1_v2v, att1_v2v,
      Wl1_v2i, bl1_v2i, Wr1_i2v, br1_i2v)


def _k1r_body(x_ref, wrvi_ref, brvi_ref, wliv_ref, bliv_ref,
              hrvi_ref, hliv_ref):
    x = x_ref[...]
    hrvi = jnp.dot(x, wrvi_ref[...].T, preferred_element_type=jnp.float32) + brvi_ref[...]
    hliv = jnp.dot(x, wliv_ref[...].T, preferred_element_type=jnp.float32) + bliv_ref[...]
    for h in range(2):
        hrvi_ref[h] = hrvi[:, 32 * h:32 * h + 32]
        hliv_ref[h] = hliv[:, 32 * h:32 * h + 32]


def _k1r(x_rsu, Wr1_v2i, br1_v2i, Wl1_i2v, bl1_i2v):
    f32 = jnp.float32
    return pl.pallas_call(
        _k1r_body,
        out_shape=[jax.ShapeDtypeStruct((2, _NR, 32), f32),
                   jax.ShapeDtypeStruct((2, _NR, 32), f32)],
    )(x_rsu, Wr1_v2i, br1_v2i, Wl1_i2v, bl1_i2v)


def _k2v_body(numvv_ref, denvv_ref, exvv_ref, hlvv_ref,
              numiv_ref, deniv_ref,
              b1vv_ref, b1iv_ref,
              wl2vv_ref, bl2vv_ref, wr2vv_ref, br2vv_ref, att2vv_ref,
              wl2vi_ref, bl2vi_ref, wr2iv_ref, br2iv_ref,
              hl2vv_ref, hr2vv_ref, ex2vv_ref, hl2vi_ref, hr2iv_ref):
    parts = []
    for h in range(2):
        ex = exvv_ref[h]
        den = denvv_ref[h] + ex
        v = (numvv_ref[h] + ex * hlvv_ref[h]) / (den + 1e-16)
        parts.append(v)
    v1 = jnp.concatenate(parts, axis=1) + b1vv_ref[...]
    parts = []
    for h in range(2):
        parts.append(numiv_ref[h] / (deniv_ref[h] + 1e-16))
    v1b = jnp.concatenate(parts, axis=1) + b1iv_ref[...]
    veh = _elu(v1 + v1b)
    hl2vv = jnp.dot(veh, wl2vv_ref[...].T, preferred_element_type=jnp.float32) + bl2vv_ref[...]
    hr2vv = jnp.dot(veh, wr2vv_ref[...].T, preferred_element_type=jnp.float32) + br2vv_ref[...]
    hl2vv_ref[...] = hl2vv
    hr2vv_ref[...] = hr2vv
    hl2vi_ref[...] = jnp.dot(veh, wl2vi_ref[...].T, preferred_element_type=jnp.float32) + bl2vi_ref[...]
    hr2iv_ref[...] = jnp.dot(veh, wr2iv_ref[...].T, preferred_element_type=jnp.float32) + br2iv_ref[...]
    ex2vv_ref[...] = jnp.exp(jnp.sum(_lrelu(hl2vv + hr2vv) * att2vv_ref[0], axis=-1))[:, None]


def _k2v(numvv, denvv, exvv, hlvv, numiv, deniv, b1vv, b1iv,
         Wl2_v2v, bl2_v2v, Wr2_v2v, br2_v2v, att2_v2v,
         Wl2_v2i, bl2_v2i, Wr2_i2v, br2_i2v):
    n = _NV
    grid = n // _BLK
    f32 = jnp.float32
    full = lambda s: pl.BlockSpec(s, lambda i: tuple(0 for _ in s))
    t3 = pl.BlockSpec((2, _BLK, 32), lambda i: (0, i, 0))
    t2 = pl.BlockSpec((2, _BLK, 1), lambda i: (0, i, 0))
    m = pl.BlockSpec((_BLK, 64), lambda i: (i, 0))
    m1 = pl.BlockSpec((_BLK, 1), lambda i: (i, 0))
    return pl.pallas_call(
        _k2v_body,
        grid=(grid,),
        in_specs=[t3, t2, t2, t3, t3, t2,
                  full((64,)), full((64,)),
                  full((64, 64)), full((64,)), full((64, 64)), full((64,)),
                  full((1, 64)),
                  full((64, 64)), full((64,)), full((64, 64)), full((64,))],
        out_specs=[m, m, m1, m, m],
        out_shape=[jax.ShapeDtypeStruct((n, 64), f32),
                   jax.ShapeDtypeStruct((n, 64), f32),
                   jax.ShapeDtypeStruct((n, 1), f32),
                   jax.ShapeDtypeStruct((n, 64), f32),
                   jax.ShapeDtypeStruct((n, 64), f32)],
    )(numvv, denvv, exvv, hlvv, numiv, deniv, b1vv, b1iv,
      Wl2_v2v, bl2_v2v, Wr2_v2v, br2_v2v, att2_v2v,
      Wl2_v2i, bl2_v2i, Wr2_i2v, br2_i2v)


def _k2r_body(numvi_ref, denvi_ref, b1vi_ref,
              wr2vi_ref, br2vi_ref, wl2iv_ref, bl2iv_ref,
              hr2vi_ref, hl2iv_ref):
    parts = []
    for h in range(2):
        parts.append(numvi_ref[h] / (denvi_ref[h] + 1e-16))
    r1 = jnp.concatenate(parts, axis=1) + b1vi_ref[...]
    rsu = _elu(r1)
    hr2vi_ref[...] = jnp.dot(rsu, wr2vi_ref[...].T, preferred_element_type=jnp.float32) + br2vi_ref[...]
    hl2iv_ref[...] = jnp.dot(rsu, wl2iv_ref[...].T, preferred_element_type=jnp.float32) + bl2iv_ref[...]


def _k2r(numvi, denvi, b1vi, Wr2_v2i, br2_v2i, Wl2_i2v, bl2_i2v):
    f32 = jnp.float32
    return pl.pallas_call(
        _k2r_body,
        out_shape=[jax.ShapeDtypeStruct((_NR, 64), f32),
                   jax.ShapeDtypeStruct((_NR, 64), f32)],
    )(numvi, denvi, b1vi, Wr2_v2i, br2_v2i, Wl2_i2v, bl2_i2v)


def _ln(x, g, b):
    mu = jnp.mean(x, axis=-1, keepdims=True)
    var = jnp.mean((x - mu) ** 2, axis=-1, keepdims=True)
    return (x - mu) * lax.rsqrt(var + 1e-5) * g + b


def _k3r_body(numvi_ref, denvi_ref, hl2vi5_ref, hr2vi_ref, attvi_ref,
              b2vi_ref, hl2iv_ref, hr2iv5_ref, attiv_ref, g_ref, b_ref,
              rsu_ref, exiv_ref):
    exvi = jnp.exp(jnp.sum(_lrelu(hl2vi5_ref[...] + hr2vi_ref[...]) * attvi_ref[0], axis=-1))[:, None]
    den = denvi_ref[...] + exvi
    r2 = (numvi_ref[...] + exvi * hl2vi5_ref[...]) / (den + 1e-16) + b2vi_ref[...]
    rsu_ref[...] = _ln(r2, g_ref[...], b_ref[...])
    exiv_ref[...] = jnp.exp(jnp.sum(_lrelu(hl2iv_ref[...] + hr2iv5_ref[...]) * attiv_ref[0], axis=-1))[:, None]


def _k3r(numvi, denvi, hl2vi5, hr2vi, att2_v2i, b2_v2i,
         hl2iv, hr2iv5, att2_i2v, g_rsu, be_rsu):
    f32 = jnp.float32
    return pl.pallas_call(
        _k3r_body,
        out_shape=[jax.ShapeDtypeStruct((_NR, 64), f32),
                   jax.ShapeDtypeStruct((_NR, 1), f32)],
    )(numvi, denvi, hl2vi5, hr2vi, att2_v2i, b2_v2i,
      hl2iv, hr2iv5, att2_i2v, g_rsu, be_rsu)


def _k3v_body(numvv_ref, denvv_ref, exvv_ref, hlvv_ref,
              numiv_ref, deniv_ref, exiv_ref, hliv_ref,
              b2vv_ref, b2iv_ref, g_ref, b_ref, out_ref):
    ex = exvv_ref[...]
    den = denvv_ref[...] + ex
    v2 = (numvv_ref[...] + ex * hlvv_ref[...]) / (den + 1e-16) + b2vv_ref[...]
    exb = exiv_ref[...]
    denb = deniv_ref[...] + exb
    v2b = (numiv_ref[...] + exb * hliv_ref[...]) / (denb + 1e-16) + b2iv_ref[...]
    out_ref[...] = _ln(v2 + v2b, g_ref[...], b_ref[...])


def _k3v(numvv, denvv, exvv, hlvv, numiv, deniv, exiv, hliv,
         b2vv, b2iv, g_veh, be_veh):
    n = _NV
    grid = n // _BLK
    f32 = jnp.float32
    full = lambda s: pl.BlockSpec(s, lambda i: tuple(0 for _ in s))
    m = pl.BlockSpec((_BLK, 64), lambda i: (i, 0))
    m1 = pl.BlockSpec((_BLK, 1), lambda i: (i, 0))
    return pl.pallas_call(
        _k3v_body,
        grid=(grid,),
        in_specs=[m, m1, m1, m, m, m1, m1, m,
                  full((64,)), full((64,)), full((64,)), full((64,))],
        out_specs=m,
        out_shape=jax.ShapeDtypeStruct((n, 64), f32),
    )(numvv, denvv, exvv, hlvv, numiv, deniv, exiv, hliv,
      b2vv, b2iv, g_veh, be_veh)


def _pad_row(t):
    h, n, ch = t.shape
    return jnp.pad(t, ((0, 0), (0, 1), (0, 0))).reshape(h * (n + 1), ch)


def _den_slice(den, acc16, own):
    return den.reshape(2, acc16 * _L)[:, :own, None]


def kernel(x_vehicle, x_rsu, edge_index_v2v, v2i_src, v2i_dst, i2v_src, i2v_dst,
           Wl1_v2v, bl1_v2v, Wr1_v2v, br1_v2v, att1_v2v, b1_v2v,
           Wl1_v2i, bl1_v2i, Wr1_v2i, br1_v2i, att1_v2i, b1_v2i,
           Wl1_i2v, bl1_i2v, Wr1_i2v, br1_i2v, att1_i2v, b1_i2v,
           Wl2_v2v, bl2_v2v, Wr2_v2v, br2_v2v, att2_v2v, b2_v2v,
           Wl2_v2i, bl2_v2i, Wr2_v2i, br2_v2i, att2_v2i, b2_v2i,
           Wl2_i2v, bl2_i2v, Wr2_i2v, br2_i2v, att2_i2v, b2_i2v,
           g_veh, be_veh, g_rsu, be_rsu):
    f32 = jnp.float32
    s, d = edge_index_v2v[0], edge_index_v2v[1]
    c32, c64 = 48, 32
    sp_vv1, dp_vv1, ep_vv1 = _pad_edges(s, d, _NV, c32)
    sp_vi1, dp_vi1, ep_vi1 = _pad_edges(v2i_src, v2i_dst, _NR, c32)
    sp_iv1, dp_iv1, ep_iv1 = _pad_edges(i2v_src, i2v_dst, _NV, c32)
    sp_vv2, dp_vv2, ep_vv2 = _pad_edges(s, d, _NV, c64)
    sp_vi2, dp_vi2, ep_vi2 = _pad_edges(v2i_src, v2i_dst, _NR, c64)
    sp_iv2, dp_iv2, ep_iv2 = _pad_edges(i2v_src, i2v_dst, _NV, c64)

    z32 = jnp.zeros((_ZC, 32), f32)
    z64 = jnp.zeros((_ZC, 64), f32)
    zd16 = jnp.zeros((_ZC, _L), f32)

    hlvv, hrvv, exvv, hlvi, hriv = _k1v(
        x_vehicle, Wl1_v2v, bl1_v2v, Wr1_v2v, br1_v2v, att1_v2v,
        Wl1_v2i, bl1_v2i, Wr1_i2v, br1_i2v)
    hrvi, hliv = _k1r(x_rsu, Wr1_v2i, br1_v2i, Wl1_i2v, bl1_i2v)

    f_vv1, acc_vv1, a16_vv1 = _make_edge_pass(ep_vv1, 32, True, _NV, _NV + 1, _NV, c32)
    f_vi1, acc_vi1, a16_vi1 = _make_edge_pass(ep_vi1, 32, True, _NV, _NR + 1, _NR, c32)
    f_iv1, acc_iv1, a16_iv1 = _make_edge_pass(ep_iv1, 32, True, _NR, _NV + 1, _NV, c32)

    num1vv, den1vv = f_vv1(sp_vv1, dp_vv1, hlvv.reshape(2 * _NV, 32),
                           _pad_row(hrvv), _attb(att1_v2v), z32, zd16)
    num1vi, den1vi = f_vi1(sp_vi1, dp_vi1, hlvi.reshape(2 * _NV, 32),
                           _pad_row(hrvi), _attb(att1_v2i), z32, zd16)
    num1iv, den1iv = f_iv1(sp_iv1, dp_iv1, hliv.reshape(2 * _NR, 32),
                           _pad_row(hriv), _attb(att1_i2v), z32, zd16)

    num1vv_s = num1vv.reshape(2, acc_vv1, 32)[:, :_NV]
    den1vv_s = _den_slice(den1vv, a16_vv1, _NV)
    num1iv_s = num1iv.reshape(2, acc_iv1, 32)[:, :_NV]
    den1iv_s = _den_slice(den1iv, a16_iv1, _NV)
    num1vi_s = num1vi.reshape(2, acc_vi1, 32)[:, :_NR]
    den1vi_s = _den_slice(den1vi, a16_vi1, _NR)

    hl2vv, hr2vv, ex2vv, hl2vi, hr2iv = _k2v(
        num1vv_s, den1vv_s, exvv, hlvv, num1iv_s, den1iv_s, b1_v2v, b1_i2v,
        Wl2_v2v, bl2_v2v, Wr2_v2v, br2_v2v, att2_v2v,
        Wl2_v2i, bl2_v2i, Wr2_i2v, br2_i2v)
    hr2vi, hl2iv = _k2r(num1vi_s, den1vi_s, b1_v2i,
                        Wr2_v2i, br2_v2i, Wl2_i2v, bl2_i2v)

    f_vv2, acc_vv2, a16_vv2 = _make_edge_pass(ep_vv2, 64, False, _NV, _NV + 1, _NV // 2, c64)
    f_vi2, acc_vi2, a16_vi2 = _make_edge_pass(ep_vi2, 64, False, _NV, _NR + 1, _NR // 2, c64)
    f_iv2, acc_iv2, a16_iv2 = _make_edge_pass(ep_iv2, 64, False, _NR, _NV + 1, _NV // 2, c64)

    hr2vv_p = jnp.pad(hr2vv, ((0, 1), (0, 0)))
    hr2vi_p = jnp.pad(hr2vi, ((0, 1), (0, 0)))
    hr2iv_p = jnp.pad(hr2iv, ((0, 1), (0, 0)))
    num2vv, den2vv = f_vv2(sp_vv2, dp_vv2, hl2vv, hr2vv_p, _attb(att2_v2v), z64, zd16)
    num2vi, den2vi = f_vi2(sp_vi2, dp_vi2, hl2vi, hr2vi_p, _attb(att2_v2i), z64, zd16)
    num2iv, den2iv = f_iv2(sp_iv2, dp_iv2, hl2iv, hr2iv_p, _attb(att2_i2v), z64, zd16)

    h2 = _NV // 2
    num2vv_s = num2vv.reshape(2, acc_vv2, 64)[:, :h2].reshape(_NV, 64)
    den2vv_s = _den_slice(den2vv, a16_vv2, h2).reshape(_NV, 1)
    num2iv_s = num2iv.reshape(2, acc_iv2, 64)[:, :h2].reshape(_NV, 64)
    den2iv_s = _den_slice(den2iv, a16_iv2, h2).reshape(_NV, 1)
    hr2n = _NR // 2
    num2vi_s = num2vi.reshape(2, acc_vi2, 64)[:, :hr2n].reshape(_NR, 64)
    den2vi_s = _den_slice(den2vi, a16_vi2, hr2n).reshape(_NR, 1)

    rsu_out, exiv = _k3r(num2vi_s, den2vi_s, hl2vi[:_NR], hr2vi, att2_v2i,
                         b2_v2i, hl2iv, hr2iv[:_NR], att2_i2v, g_rsu, be_rsu)
    exiv_pad = jnp.pad(exiv, ((0, _NV - _NR), (0, 0)))
    hl2iv_pad = jnp.pad(hl2iv, ((0, _NV - _NR), (0, 0)))
    veh_out = _k3v(num2vv_s, den2vv_s, ex2vv, hl2vv,
                   num2iv_s, den2iv_s, exiv_pad, hl2iv_pad,
                   b2_v2v, b2_i2v, g_veh, be_veh)
    return veh_out, rsu_out
